# baseline (speedup 1.0000x reference)
import functools

import jax
import jax.numpy as jnp
import numpy as np
from jax import lax
from jax.experimental import pallas as pl
from jax.experimental.pallas import tpu as pltpu
from jax.experimental.pallas import tpu_sc as plsc

_PROB = 0.12
_T = 4096
_ROWS = 16 * 256
_NW = 32
_ROWS_PER_TILE = _ROWS // _NW
_RB = 8
_BUFW = _RB * _T
_NBLK = _ROWS_PER_TILE // _RB
_NBUF = 3
_L = 16


def _rotl(x, d):
    return ((x << np.uint32(d)) | (x >> np.uint32(32 - d))).astype(np.uint32)


def _threefry2x32_core(ks0, ks1, x0, x1):
    ks2 = np.uint32(ks0 ^ ks1 ^ np.uint32(0x1BD11BDA))
    rot = [[13, 15, 26, 6], [17, 29, 16, 24]]
    x0 = (x0 + ks0).astype(np.uint32)
    x1 = (x1 + ks1).astype(np.uint32)
    inject = [(ks1, ks2), (ks2, ks0), (ks0, ks1), (ks1, ks2), (ks2, ks0)]
    for g in range(5):
        for r in rot[g % 2]:
            x0 = (x0 + x1).astype(np.uint32)
            x1 = _rotl(x1, r)
            x1 = (x1 ^ x0).astype(np.uint32)
        a, b = inject[g]
        x0 = (x0 + a).astype(np.uint32)
        x1 = (x1 + b + np.uint32(g + 1)).astype(np.uint32)
    return x0, x1


def _uniform01(ks0, ks1, n):
    b1, b2 = _threefry2x32_core(
        ks0, ks1, np.zeros(n, dtype=np.uint32), np.arange(n, dtype=np.uint32))
    bits = (b1 ^ b2).astype(np.uint32)
    return ((bits >> np.uint32(9)) | np.uint32(0x3F800000)).view(np.float32) - np.float32(1.0)


def _jitter_index_constants():
    b1, b2 = _threefry2x32_core(np.uint32(0), np.uint32(42),
                                np.zeros(2, dtype=np.uint32),
                                np.arange(2, dtype=np.uint32))
    replace = _uniform01(b1[0], b2[0], _T) < np.float32(_PROB)
    direction = np.where(_uniform01(b1[1], b2[1], _T) < np.float32(0.5), -1, 1)
    i = np.arange(_T)
    offset = np.where(i == 0, 1, np.where(i == _T - 1, -1, direction))
    final = np.where(replace, i + offset, i)
    pos = np.nonzero(final != i)[0]
    nb = final[pos]
    return pos.astype(np.int32), nb.astype(np.int32)


_POS, _NB = _jitter_index_constants()
_NREAL = len(_POS)


def _split_fixups():
    pos, nb = _POS[: _NREAL], _NB[: _NREAL]
    replaced = np.zeros(_T, dtype=bool)
    replaced[pos] = True
    danger = replaced[nb]
    return pos[danger], nb[danger], pos[~danger], nb[~danger]


def _pack_blk(pos, nb):
    npad = (-len(pos)) % _L
    pos = np.concatenate([pos, np.repeat(pos[:1], npad)]).astype(np.int64)
    nb = np.concatenate([nb, np.repeat(nb[:1], npad)]).astype(np.int64)
    row = np.repeat(np.arange(_RB, dtype=np.int64), len(pos))
    posb = np.tile(pos, _RB)
    nbb = np.tile(nb, _RB)
    return (row * 8192 + posb * 2 + (nbb > posb)).astype(np.int32)


_DPOS, _DNB, _SPOS, _SNB = _split_fixups()
_DPACK_B = _pack_blk(_DPOS, _DNB)
_SPACK_B = _pack_blk(_SPOS, _SNB)
_NDB = len(_DPACK_B)
_NSB = len(_SPACK_B)
_NDC = _NDB // _L
_NSC = _NSB // _L


def _sc_jitter(x2d, dpack, spack):
    mesh = plsc.VectorSubcoreMesh(core_axis_name="c", subcore_axis_name="s")

    @functools.partial(
        pl.kernel,
        mesh=mesh,
        out_type=jax.ShapeDtypeStruct((_ROWS, _T), jnp.float32),
        compiler_params=pltpu.CompilerParams(needs_layout_passes=False),
        scratch_types=[
            pltpu.VMEM((_NDB,), jnp.int32),
            pltpu.VMEM((_NSB,), jnp.int32),
            pltpu.VMEM((_NDB,), jnp.float32),
            [pltpu.VMEM((_RB, _T), jnp.float32)] * _NBUF,
            [pltpu.SemaphoreType.DMA] * _NBUF,
            [pltpu.SemaphoreType.DMA] * _NBUF,
        ],
    )
    def k(x_hbm, dpack_hbm, spack_hbm, out_hbm, dpack_v, spack_v, gat_v,
          bufs, sins, souts):
        wid = lax.axis_index("s") * 2 + lax.axis_index("c")
        pltpu.sync_copy(dpack_hbm, dpack_v)
        pltpu.sync_copy(spack_hbm, spack_v)
        tile_row = wid * _ROWS_PER_TILE

        def unpack(p):
            row = lax.shift_right_logical(p, 13)
            pos = lax.shift_right_logical(p, 1) & 4095
            nb = pos + (p & 1) * 2 - 1
            return row, pos, nb

        def fixup(buf):
            def dgather_body(j, c):
                sl = pl.ds(j * _L, _L)
                row, _, nb = unpack(dpack_v[sl])
                gat_v[sl] = plsc.load_gather(buf, [row, nb])
                return c

            lax.fori_loop(0, _NDC, dgather_body, 0, unroll=8)

            def dscatter_body(j, c):
                sl = pl.ds(j * _L, _L)
                row, pos, _ = unpack(dpack_v[sl])
                plsc.store_scatter(buf, [row, pos], gat_v[sl])
                return c

            lax.fori_loop(0, _NDC, dscatter_body, 0, unroll=8)

            def sfused_body(j, c):
                sl = pl.ds(j * _L, _L)
                row, pos, nb = unpack(spack_v[sl])
                g = plsc.load_gather(buf, [row, nb])
                plsc.store_scatter(buf, [row, pos], g)
                return c

            lax.fori_loop(0, _NSC, sfused_body, 0, unroll=8)

        def start_in(b, buf_i):
            return pltpu.async_copy(
                x_hbm.at[pl.ds(tile_row + b * _RB, _RB), :],
                bufs[buf_i], sins[buf_i])

        h_in = [None] * _NBUF
        h_out = [None] * _NBUF
        h_in[0] = start_in(0, 0)
        for b in range(_NBLK):
            cur = b % _NBUF
            ahead = (b + 1) % _NBUF
            if b + 1 < _NBLK:
                if h_out[ahead] is not None:
                    h_out[ahead].wait()
                h_in[ahead] = start_in(b + 1, ahead)
            h_in[cur].wait()
            fixup(bufs[cur])
            h_out[cur] = pltpu.async_copy(
                bufs[cur], out_hbm.at[pl.ds(tile_row + b * _RB, _RB), :],
                souts[cur])
        for p in range(_NBUF):
            if h_out[p] is not None:
                h_out[p].wait()

    return k(x2d, dpack, spack)


def kernel(quantized):
    B, C, T = quantized.shape
    x2d = quantized.reshape(B * C, T)
    out = _sc_jitter(x2d, jnp.asarray(_DPACK_B), jnp.asarray(_SPACK_B))
    return out.reshape(B, C, T)


# speedup vs baseline: 1.0453x; 1.0453x over previous
import functools

import jax
import jax.numpy as jnp
import numpy as np
from jax import lax
from jax.experimental import pallas as pl
from jax.experimental.pallas import tpu as pltpu
from jax.experimental.pallas import tpu_sc as plsc

_PROB = 0.12
_T = 4096
_ROWS = 16 * 256
_NW = 32
_ROWS_PER_TILE = _ROWS // _NW
_RB = 8
_BUFW = _RB * _T
_NBLK = _ROWS_PER_TILE // _RB
_NBUF = 3
_L = 16


def _rotl(x, d):
    return ((x << np.uint32(d)) | (x >> np.uint32(32 - d))).astype(np.uint32)


def _threefry2x32_core(ks0, ks1, x0, x1):
    ks2 = np.uint32(ks0 ^ ks1 ^ np.uint32(0x1BD11BDA))
    rot = [[13, 15, 26, 6], [17, 29, 16, 24]]
    x0 = (x0 + ks0).astype(np.uint32)
    x1 = (x1 + ks1).astype(np.uint32)
    inject = [(ks1, ks2), (ks2, ks0), (ks0, ks1), (ks1, ks2), (ks2, ks0)]
    for g in range(5):
        for r in rot[g % 2]:
            x0 = (x0 + x1).astype(np.uint32)
            x1 = _rotl(x1, r)
            x1 = (x1 ^ x0).astype(np.uint32)
        a, b = inject[g]
        x0 = (x0 + a).astype(np.uint32)
        x1 = (x1 + b + np.uint32(g + 1)).astype(np.uint32)
    return x0, x1


def _uniform01(ks0, ks1, n):
    b1, b2 = _threefry2x32_core(
        ks0, ks1, np.zeros(n, dtype=np.uint32), np.arange(n, dtype=np.uint32))
    bits = (b1 ^ b2).astype(np.uint32)
    return ((bits >> np.uint32(9)) | np.uint32(0x3F800000)).view(np.float32) - np.float32(1.0)


def _jitter_index_constants():
    b1, b2 = _threefry2x32_core(np.uint32(0), np.uint32(42),
                                np.zeros(2, dtype=np.uint32),
                                np.arange(2, dtype=np.uint32))
    replace = _uniform01(b1[0], b2[0], _T) < np.float32(_PROB)
    direction = np.where(_uniform01(b1[1], b2[1], _T) < np.float32(0.5), -1, 1)
    i = np.arange(_T)
    offset = np.where(i == 0, 1, np.where(i == _T - 1, -1, direction))
    final = np.where(replace, i + offset, i)
    pos = np.nonzero(final != i)[0]
    nb = final[pos]
    return pos.astype(np.int32), nb.astype(np.int32)


_POS, _NB = _jitter_index_constants()
_NREAL = len(_POS)


def _split_fixups():
    pos, nb = _POS[: _NREAL], _NB[: _NREAL]
    replaced = np.zeros(_T, dtype=bool)
    replaced[pos] = True
    danger = replaced[nb]
    return pos[danger], nb[danger], pos[~danger], nb[~danger]


def _blk_arrays(pos, nb):
    npad = (-len(pos)) % _L
    pos = np.concatenate([pos, np.repeat(pos[:1], npad)]).astype(np.int32)
    nb = np.concatenate([nb, np.repeat(nb[:1], npad)]).astype(np.int32)
    row = np.repeat(np.arange(_RB, dtype=np.int32), len(pos))
    return row, np.tile(pos, _RB), np.tile(nb, _RB)


_DPOS, _DNB, _SPOS, _SNB = _split_fixups()
_DROW_B, _DPOS_B, _DNB_B = _blk_arrays(_DPOS, _DNB)
_SROW_B, _SPOS_B, _SNB_B = _blk_arrays(_SPOS, _SNB)
_NDB = len(_DPOS_B)
_NSB = len(_SPOS_B)
_NDC = _NDB // _L
_NSC = _NSB // _L


def _sc_jitter(x2d, drow, dpos, dnb, srow, spos, snb):
    mesh = plsc.VectorSubcoreMesh(core_axis_name="c", subcore_axis_name="s")

    @functools.partial(
        pl.kernel,
        mesh=mesh,
        out_type=jax.ShapeDtypeStruct((_ROWS, _T), jnp.float32),
        compiler_params=pltpu.CompilerParams(needs_layout_passes=False),
        scratch_types=[
            pltpu.VMEM((_NDB,), jnp.int32),
            pltpu.VMEM((_NDB,), jnp.int32),
            pltpu.VMEM((_NDB,), jnp.int32),
            pltpu.VMEM((_NSB,), jnp.int32),
            pltpu.VMEM((_NSB,), jnp.int32),
            pltpu.VMEM((_NSB,), jnp.int32),
            pltpu.VMEM((_NDB,), jnp.float32),
            [pltpu.VMEM((_RB, _T), jnp.float32)] * _NBUF,
            [pltpu.SemaphoreType.DMA] * _NBUF,
            [pltpu.SemaphoreType.DMA] * _NBUF,
        ],
    )
    def k(x_hbm, drow_hbm, dpos_hbm, dnb_hbm, srow_hbm, spos_hbm, snb_hbm,
          out_hbm, drow_v, dpos_v, dnb_v, srow_v, spos_v, snb_v, gat_v,
          bufs, sins, souts):
        wid = lax.axis_index("s") * 2 + lax.axis_index("c")
        pltpu.sync_copy(drow_hbm, drow_v)
        pltpu.sync_copy(dpos_hbm, dpos_v)
        pltpu.sync_copy(dnb_hbm, dnb_v)
        pltpu.sync_copy(srow_hbm, srow_v)
        pltpu.sync_copy(spos_hbm, spos_v)
        pltpu.sync_copy(snb_hbm, snb_v)
        tile_row = wid * _ROWS_PER_TILE

        def fixup(buf):
            def dgather_body(j, c):
                sl = pl.ds(j * _L, _L)
                gat_v[sl] = plsc.load_gather(buf, [drow_v[sl], dnb_v[sl]])
                return c

            lax.fori_loop(0, _NDC, dgather_body, 0, unroll=8)

            def dscatter_body(j, c):
                sl = pl.ds(j * _L, _L)
                plsc.store_scatter(buf, [drow_v[sl], dpos_v[sl]], gat_v[sl])
                return c

            lax.fori_loop(0, _NDC, dscatter_body, 0, unroll=8)

            def sfused_body(j, c):
                sl = pl.ds(j * _L, _L)
                g = plsc.load_gather(buf, [srow_v[sl], snb_v[sl]])
                plsc.store_scatter(buf, [srow_v[sl], spos_v[sl]], g)
                return c

            lax.fori_loop(0, _NSC, sfused_body, 0, unroll=12)

        def start_in(b, buf_i):
            return pltpu.async_copy(
                x_hbm.at[pl.ds(tile_row + b * _RB, _RB), :],
                bufs[buf_i], sins[buf_i])

        h_in = [None] * _NBUF
        h_out = [None] * _NBUF
        h_in[0] = start_in(0, 0)
        for b in range(_NBLK):
            cur = b % _NBUF
            ahead = (b + 1) % _NBUF
            if b + 1 < _NBLK:
                if h_out[ahead] is not None:
                    h_out[ahead].wait()
                h_in[ahead] = start_in(b + 1, ahead)
            h_in[cur].wait()
            fixup(bufs[cur])
            h_out[cur] = pltpu.async_copy(
                bufs[cur], out_hbm.at[pl.ds(tile_row + b * _RB, _RB), :],
                souts[cur])
        for p in range(_NBUF):
            if h_out[p] is not None:
                h_out[p].wait()

    return k(x2d, drow, dpos, dnb, srow, spos, snb)


def kernel(quantized):
    B, C, T = quantized.shape
    x2d = quantized.reshape(B * C, T)
    out = _sc_jitter(x2d, jnp.asarray(_DROW_B), jnp.asarray(_DPOS_B),
                     jnp.asarray(_DNB_B), jnp.asarray(_SROW_B),
                     jnp.asarray(_SPOS_B), jnp.asarray(_SNB_B))
    return out.reshape(B, C, T)


# speedup vs baseline: 1.2627x; 1.2079x over previous
import functools

import jax
import jax.numpy as jnp
import numpy as np
from jax import lax
from jax.experimental import pallas as pl
from jax.experimental.pallas import tpu as pltpu
from jax.experimental.pallas import tpu_sc as plsc

_PROB = 0.12
_T = 4096
_ROWS = 16 * 256
_NW = 32
_ROWS_PER_TILE = _ROWS // _NW
_RB = 8
_BUFW = _RB * _T
_NBLK = _ROWS_PER_TILE // _RB
_NBUF = 3
_L = 16


def _rotl(x, d):
    return ((x << np.uint32(d)) | (x >> np.uint32(32 - d))).astype(np.uint32)


def _threefry2x32_core(ks0, ks1, x0, x1):
    ks2 = np.uint32(ks0 ^ ks1 ^ np.uint32(0x1BD11BDA))
    rot = [[13, 15, 26, 6], [17, 29, 16, 24]]
    x0 = (x0 + ks0).astype(np.uint32)
    x1 = (x1 + ks1).astype(np.uint32)
    inject = [(ks1, ks2), (ks2, ks0), (ks0, ks1), (ks1, ks2), (ks2, ks0)]
    for g in range(5):
        for r in rot[g % 2]:
            x0 = (x0 + x1).astype(np.uint32)
            x1 = _rotl(x1, r)
            x1 = (x1 ^ x0).astype(np.uint32)
        a, b = inject[g]
        x0 = (x0 + a).astype(np.uint32)
        x1 = (x1 + b + np.uint32(g + 1)).astype(np.uint32)
    return x0, x1


def _uniform01(ks0, ks1, n):
    b1, b2 = _threefry2x32_core(
        ks0, ks1, np.zeros(n, dtype=np.uint32), np.arange(n, dtype=np.uint32))
    bits = (b1 ^ b2).astype(np.uint32)
    return ((bits >> np.uint32(9)) | np.uint32(0x3F800000)).view(np.float32) - np.float32(1.0)


def _jitter_index_constants():
    b1, b2 = _threefry2x32_core(np.uint32(0), np.uint32(42),
                                np.zeros(2, dtype=np.uint32),
                                np.arange(2, dtype=np.uint32))
    replace = _uniform01(b1[0], b2[0], _T) < np.float32(_PROB)
    direction = np.where(_uniform01(b1[1], b2[1], _T) < np.float32(0.5), -1, 1)
    i = np.arange(_T)
    offset = np.where(i == 0, 1, np.where(i == _T - 1, -1, direction))
    final = np.where(replace, i + offset, i)
    pos = np.nonzero(final != i)[0]
    nb = final[pos]
    return pos.astype(np.int32), nb.astype(np.int32)


_POS, _NB = _jitter_index_constants()
_NREAL = len(_POS)


def _split_fixups():
    pos, nb = _POS[: _NREAL], _NB[: _NREAL]
    replaced = np.zeros(_T, dtype=bool)
    replaced[pos] = True
    danger = replaced[nb]
    return pos[danger], nb[danger], pos[~danger], nb[~danger]


def _blk_arrays(pos, nb):
    npad = (-len(pos)) % _L
    pos = np.concatenate([pos, np.repeat(pos[:1], npad)]).astype(np.int32)
    nb = np.concatenate([nb, np.repeat(nb[:1], npad)]).astype(np.int32)
    row = np.repeat(np.arange(_RB, dtype=np.int32), len(pos))
    return row, np.tile(pos, _RB), np.tile(nb, _RB)


_DPOS, _DNB, _SPOS, _SNB = _split_fixups()
_DROW_B, _DPOS_B, _DNB_B = _blk_arrays(_DPOS, _DNB)
_SROW_B, _SPOS_B, _SNB_B = _blk_arrays(_SPOS, _SNB)
_NDB = len(_DPOS_B)
_NSB = len(_SPOS_B)
_NDC = _NDB // _L
_NSC = _NSB // _L


def _sc_jitter(x2d, drow, dpos, dnb, srow, spos, snb):
    mesh = plsc.VectorSubcoreMesh(core_axis_name="c", subcore_axis_name="s")

    @functools.partial(
        pl.kernel,
        mesh=mesh,
        out_type=jax.ShapeDtypeStruct((_ROWS, _T), jnp.float32),
        compiler_params=pltpu.CompilerParams(needs_layout_passes=False),
        scratch_types=[
            pltpu.VMEM((_NDB,), jnp.int32),
            pltpu.VMEM((_NDB,), jnp.int32),
            pltpu.VMEM((_NDB,), jnp.int32),
            pltpu.VMEM((_NSB,), jnp.int32),
            pltpu.VMEM((_NSB,), jnp.int32),
            pltpu.VMEM((_NSB,), jnp.int32),
            pltpu.VMEM((_NDB,), jnp.float32),
            [pltpu.VMEM((_RB, _T), jnp.float32)] * _NBUF,
            [pltpu.SemaphoreType.DMA] * _NBUF,
            [pltpu.SemaphoreType.DMA] * _NBUF,
        ],
    )
    def k(x_hbm, drow_hbm, dpos_hbm, dnb_hbm, srow_hbm, spos_hbm, snb_hbm,
          out_hbm, drow_v, dpos_v, dnb_v, srow_v, spos_v, snb_v, gat_v,
          bufs, sins, souts):
        wid = lax.axis_index("s") * 2 + lax.axis_index("c")
        pltpu.sync_copy(drow_hbm, drow_v)
        pltpu.sync_copy(dpos_hbm, dpos_v)
        pltpu.sync_copy(dnb_hbm, dnb_v)
        pltpu.sync_copy(srow_hbm, srow_v)
        pltpu.sync_copy(spos_hbm, spos_v)
        pltpu.sync_copy(snb_hbm, snb_v)
        tile_row = wid * _ROWS_PER_TILE

        def fixup(buf):
            @plsc.parallel_loop(0, _NDC, unroll=8)
            def dgather_body(j):
                sl = pl.ds(j * _L, _L)
                gat_v[sl] = plsc.load_gather(buf, [drow_v[sl], dnb_v[sl]])

            @plsc.parallel_loop(0, _NDC, unroll=8)
            def dscatter_body(j):
                sl = pl.ds(j * _L, _L)
                plsc.store_scatter(buf, [drow_v[sl], dpos_v[sl]], gat_v[sl])

            @plsc.parallel_loop(0, _NSC, unroll=12)
            def sfused_body(j):
                sl = pl.ds(j * _L, _L)
                g = plsc.load_gather(buf, [srow_v[sl], snb_v[sl]])
                plsc.store_scatter(buf, [srow_v[sl], spos_v[sl]], g)

        def start_in(b, buf_i):
            return pltpu.async_copy(
                x_hbm.at[pl.ds(tile_row + b * _RB, _RB), :],
                bufs[buf_i], sins[buf_i])

        h_in = [None] * _NBUF
        h_out = [None] * _NBUF
        h_in[0] = start_in(0, 0)
        for b in range(_NBLK):
            cur = b % _NBUF
            ahead = (b + 1) % _NBUF
            if b + 1 < _NBLK:
                if h_out[ahead] is not None:
                    h_out[ahead].wait()
                h_in[ahead] = start_in(b + 1, ahead)
            h_in[cur].wait()
            fixup(bufs[cur])
            h_out[cur] = pltpu.async_copy(
                bufs[cur], out_hbm.at[pl.ds(tile_row + b * _RB, _RB), :],
                souts[cur])
        for p in range(_NBUF):
            if h_out[p] is not None:
                h_out[p].wait()

    return k(x2d, drow, dpos, dnb, srow, spos, snb)


def kernel(quantized):
    B, C, T = quantized.shape
    x2d = quantized.reshape(B * C, T)
    out = _sc_jitter(x2d, jnp.asarray(_DROW_B), jnp.asarray(_DPOS_B),
                     jnp.asarray(_DNB_B), jnp.asarray(_SROW_B),
                     jnp.asarray(_SPOS_B), jnp.asarray(_SNB_B))
    return out.reshape(B, C, T)
